# baseline (speedup 1.0000x reference)
import functools

import jax
import jax.numpy as jnp
from jax import lax
from jax.experimental import pallas as pl
from jax.experimental.pallas import tpu as pltpu
from jax.experimental.pallas import tpu_sc as plsc

_NC, _NS = 2, 16
_NW = _NC * _NS


def _sc_add(x2, w):
    R, D = x2.shape
    L, _ = w.shape
    B = R // L
    seq_per_w = L // _NW
    CH = 4
    NCH = seq_per_w // CH
    XR = B * CH

    mesh = plsc.VectorSubcoreMesh(core_axis_name="c", subcore_axis_name="s")

    @functools.partial(
        pl.kernel,
        mesh=mesh,
        out_type=jax.ShapeDtypeStruct((R, D), jnp.float32),
        scratch_types=[
            pltpu.VMEM((XR, D), jnp.float32),
            pltpu.VMEM((XR, D), jnp.float32),
            pltpu.VMEM((CH, D), jnp.float32),
            pltpu.VMEM((CH, D), jnp.float32),
            pltpu.VMEM((XR, D), jnp.float32),
            pltpu.VMEM((XR, D), jnp.float32),
            pltpu.SemaphoreType.DMA,
            pltpu.SemaphoreType.DMA,
            pltpu.SemaphoreType.DMA,
            pltpu.SemaphoreType.DMA,
            pltpu.SemaphoreType.DMA,
            pltpu.SemaphoreType.DMA,
        ],
    )
    def run(x_hbm, w_hbm, o_hbm, xA, xB, wA, wB, oA, oB,
            sxA, sxB, swA, swB, soA, soB):
        wid = lax.axis_index("s") * _NC + lax.axis_index("c")
        sbase = wid * seq_per_w

        def in_copies(k, xbuf, wbuf, sx, sw):
            cps = [
                pltpu.make_async_copy(
                    w_hbm.at[pl.ds(sbase + k * CH, CH)], wbuf, sw
                )
            ]
            for b in range(B):
                cps.append(
                    pltpu.make_async_copy(
                        x_hbm.at[pl.ds(b * L + sbase + k * CH, CH)],
                        xbuf.at[pl.ds(b * CH, CH)],
                        sx,
                    )
                )
            return cps

        def out_copies(k, obuf, so):
            return [
                pltpu.make_async_copy(
                    obuf.at[pl.ds(b * CH, CH)],
                    o_hbm.at[pl.ds(b * L + sbase + k * CH, CH)],
                    so,
                )
                for b in range(B)
            ]

        for cp in in_copies(0, xA, wA, sxA, swA):
            cp.start()
        for cp in in_copies(1, xB, wB, sxB, swB):
            cp.start()

        def step(k, xbuf, wbuf, obuf, sx, sw, so):
            for cp in in_copies(k, xbuf, wbuf, sx, sw):
                cp.wait()

            @pl.when(k >= 2)
            def _():
                for cp in out_copies(k - 2, obuf, so):
                    cp.wait()

            @pl.loop(0, CH)
            def _(r):
                rs = pl.ds(r, 1)
                for c in range(0, D, 16):
                    cs = pl.ds(c, 16)
                    wv = wbuf.at[rs, cs][...]
                    for b in range(B):
                        rb = pl.ds(b * CH + r, 1)
                        obuf.at[rb, cs][...] = xbuf.at[rb, cs][...] + wv

            for cp in out_copies(k, obuf, so):
                cp.start()

            @pl.when(k + 2 < NCH)
            def _():
                for cp in in_copies(k + 2, xbuf, wbuf, sx, sw):
                    cp.start()

        @pl.loop(0, NCH, step=2)
        def _(k):
            step(k, xA, wA, oA, sxA, swA, soA)
            step(k + 1, xB, wB, oB, sxB, swB, soB)

        for cp in out_copies(NCH - 2, oA, soA):
            cp.wait()
        for cp in out_copies(NCH - 1, oB, soB):
            cp.wait()

    return run(x2, w)


def kernel(x, weight):
    B, L, D = x.shape
    out2 = _sc_add(x.reshape(B * L, D), weight[:L])
    return out2.reshape(B, L, D)


# speedup vs baseline: 3.4452x; 3.4452x over previous
import jax
import jax.numpy as jnp
from jax.experimental import pallas as pl
from jax.experimental.pallas import tpu as pltpu


def _add_body(x_ref, w_ref, o_ref, BL, L):
    start = (pl.program_id(0) * BL) % L
    o_ref[...] = x_ref[...] + w_ref[pl.ds(start, BL), :]


def kernel(x, weight):
    import functools

    B, L, D = x.shape
    w = weight[:L]
    x2 = x.reshape(B * L, D)
    BL = 2048
    grid = ((B * L) // BL,)
    out2 = pl.pallas_call(
        functools.partial(_add_body, BL=BL, L=L),
        grid=grid,
        in_specs=[
            pl.BlockSpec((BL, D), lambda i: (i, 0)),
            pl.BlockSpec((L, D), lambda i: (0, 0)),
        ],
        out_specs=pl.BlockSpec((BL, D), lambda i: (i, 0)),
        out_shape=jax.ShapeDtypeStruct((B * L, D), x.dtype),
        compiler_params=pltpu.CompilerParams(
            dimension_semantics=("arbitrary",),
        ),
    )(x2, w)
    return out2.reshape(B, L, D)
